# baseline (speedup 1.0000x reference)
import jax
import jax.numpy as jnp
import numpy as np
from jax.experimental import pallas as pl
from jax.experimental.pallas import tpu as pltpu

N, V, T = 64, 512, 16
K = 8
LATENT = 64
E = 4
EPS = np.float32(1e-8)
ZERO_NORM = np.float32(np.sqrt(np.float32(1e-8)))
BIG = np.float32(1e9)


def _relu(x):
    return jnp.maximum(x, 0.0)


def _dot(a, b):
    return jax.lax.dot_general(a, b, (((1,), (0,)), ((), ())),
                               preferred_element_type=jnp.float32)


def _one_env(gp, gv, tp,
             Wenc1, benc1, Wenc2, benc2, We1, be1, We2, be2,
             Wmsg, bmsg, Wupd, bupd, Wdec1, bdec1, Wdec2, bdec2,
             dist_ref, idx_ref):
    tm = jnp.mean(tp, axis=0, keepdims=True)
    corr = tm - gp

    X2 = jnp.concatenate([gp, corr, gv], axis=1)
    X2 = X2 * X2
    Sr = jax.lax.broadcasted_iota(jnp.int32, (9, 3), 0)
    Sc = jax.lax.broadcasted_iota(jnp.int32, (9, 3), 1)
    S = ((Sr >= 3 * Sc) & (Sr < 3 * Sc + 3)).astype(jnp.float32)
    n2all = _dot(X2, S)
    nall = jnp.sqrt(n2all + EPS)

    G = jax.lax.dot_general(gp, gp, (((1,), (1,)), ((), ())),
                            preferred_element_type=jnp.float32)
    rowio = jax.lax.broadcasted_iota(jnp.int32, (V, V), 0)
    colio = jax.lax.broadcasted_iota(jnp.int32, (V, V), 1)
    diag = rowio == colio
    n2col = n2all[:, 0:1]
    n2row = jnp.sum(jnp.where(diag, G, 0.0), axis=0, keepdims=True)
    d2 = n2col + n2row - 2.0 * G
    d2 = jnp.where(diag, BIG, d2)

    feat = jnp.concatenate([
        jnp.ones((V, 1), jnp.float32), nall,
        jnp.full((V, 1), ZERO_NORM, jnp.float32),
    ], axis=1)

    h = _dot(_relu(_dot(feat, Wenc1[...]) + benc1[...]), Wenc2[...]) + benc2[...]

    _i32 = lambda x: jax.lax.bitcast_convert_type(x, jnp.int32)
    _f32 = lambda x: jax.lax.bitcast_convert_type(x, jnp.float32)
    d2c = jnp.maximum(d2, jnp.float32(1e-30))
    pk = _f32((_i32(d2c) & ~jnp.int32(V - 1)) | colio)
    for k in range(K):
        kmin = _i32(jnp.min(pk, axis=1, keepdims=True))
        idx = kmin & jnp.int32(V - 1)
        d2sel = _f32(kmin & ~jnp.int32(V - 1))
        idx_ref[pl.ds(k * V, V), :] = idx
        dist_ref[pl.ds(k * V, V), :] = d2sel
        if k < K - 1:
            pk = jnp.where(colio == idx, jnp.float32(jnp.inf), pk)

    idx_all = idx_ref[...]
    cb = jax.lax.broadcasted_iota(jnp.int32, (K * V, V), 1)
    ohe = (cb == idx_all).astype(jnp.float32)

    g1 = _dot(ohe, jnp.concatenate([h, gp], axis=1))
    hs = g1[:, :LATENT]
    gsrc = g1[:, LATENT:]

    gdst = jnp.concatenate([gp] * K, axis=0)
    rel = gsrc - gdst
    dist = jnp.sqrt(dist_ref[...] + EPS)
    e_in = jnp.concatenate([rel, dist], axis=1)
    ea = _dot(_relu(_dot(e_in, We1[...]) + be1[...]), We2[...]) + be2[...]

    for r in range(2):
        if r:
            hs = _dot(ohe, h)
        msg = _relu(_dot(hs * ea, Wmsg[...]) + bmsg[...])
        agg = msg[0 * V:1 * V]
        for k in range(1, K):
            agg = agg + msg[k * V:(k + 1) * V]
        h = h + _relu(_dot(agg, Wupd[...]) + bupd[...])

    z = _dot(_relu(_dot(h, Wdec1[...]) + bdec1[...]), Wdec2[...]) + bdec2[...]
    return jnp.mean(z, axis=0, keepdims=True)


def _env_body(gp_ref, gv_ref, tp_ref,
              Wenc1, benc1, Wenc2, benc2, We1, be1, We2, be2,
              Wmsg, bmsg, Wupd, bupd, Wdec1, bdec1, Wdec2, bdec2,
              out_ref, dist_ref, idx_ref):
    for e in range(E):
        out_ref[e] = _one_env(
            gp_ref[e], gv_ref[e], tp_ref[e],
            Wenc1, benc1, Wenc2, benc2, We1, be1, We2, be2,
            Wmsg, bmsg, Wupd, bupd, Wdec1, bdec1, Wdec2, bdec2,
            dist_ref, idx_ref)


@jax.jit
def _run(gp, gv, tp, Wenc1, benc1, Wenc2, benc2, We1, be1, We2, be2,
         Wmsg, bmsg, Wupd, bupd, Wdec1, bdec1, Wdec2, bdec2):
    env_spec3 = lambda s: pl.BlockSpec((E,) + s, lambda i: (i, 0, 0))
    wspec = lambda s: pl.BlockSpec(s, lambda i: (0, 0))
    args = (Wenc1, benc1, Wenc2, benc2, We1, be1, We2, be2,
            Wmsg, bmsg, Wupd, bupd, Wdec1, bdec1, Wdec2, bdec2)
    out = pl.pallas_call(
        _env_body,
        grid=(N // E,),
        in_specs=[env_spec3((V, 3)), env_spec3((V, 3)), env_spec3((T, 3))]
                 + [wspec(a.shape) for a in args],
        out_specs=pl.BlockSpec((E, 1, LATENT), lambda i: (i, 0, 0)),
        out_shape=jax.ShapeDtypeStruct((N, 1, LATENT), jnp.float32),
        scratch_shapes=[
            pltpu.VMEM((K * V, 1), jnp.float32),
            pltpu.VMEM((K * V, 1), jnp.int32),
        ],
    )(gp, gv, tp, *args)
    return out.reshape(N, LATENT)


def kernel(glove_pos, glove_vel, ee_pos, ee_vel, target_pos,
           W_enc1, b_enc1, W_enc2, b_enc2, W_e1, b_e1, W_e2, b_e2,
           W_msg, b_msg, W_upd, b_upd, W_dec1, b_dec1, W_dec2, b_dec2):
    del ee_pos, ee_vel
    b2 = lambda b: b.reshape(1, LATENT)
    return _run(glove_pos, glove_vel, target_pos,
                W_enc1, b2(b_enc1), W_enc2, b2(b_enc2),
                W_e1, b2(b_e1), W_e2, b2(b_e2),
                W_msg, b2(b_msg), W_upd, b2(b_upd),
                W_dec1, b2(b_dec1), W_dec2, b2(b_dec2))


# speedup vs baseline: 1.0321x; 1.0321x over previous
import jax
import jax.numpy as jnp
import numpy as np
from jax.experimental import pallas as pl
from jax.experimental.pallas import tpu as pltpu

N, V, T = 64, 512, 16
K = 8
LATENT = 64
E = 1
EPS = np.float32(1e-8)
ZERO_NORM = np.float32(np.sqrt(np.float32(1e-8)))
BIG = np.float32(1e9)


def _relu(x):
    return jnp.maximum(x, 0.0)


def _dot(a, b):
    return jax.lax.dot_general(a, b, (((1,), (0,)), ((), ())),
                               preferred_element_type=jnp.float32)


def _one_env(gp, gv, tp,
             Wenc1, benc1, Wenc2, benc2, We1, be1, We2, be2,
             Wmsg, bmsg, Wupd, bupd, Wdec1, bdec1, Wdec2, bdec2,
             dist_ref, idx_ref):
    tm = jnp.mean(tp, axis=0, keepdims=True)
    corr = tm - gp

    X2 = jnp.concatenate([gp, corr, gv], axis=1)
    X2 = X2 * X2
    Sr = jax.lax.broadcasted_iota(jnp.int32, (9, 3), 0)
    Sc = jax.lax.broadcasted_iota(jnp.int32, (9, 3), 1)
    S = ((Sr >= 3 * Sc) & (Sr < 3 * Sc + 3)).astype(jnp.float32)
    n2all = _dot(X2, S)
    nall = jnp.sqrt(n2all + EPS)

    G = jax.lax.dot_general(gp, gp, (((1,), (1,)), ((), ())),
                            preferred_element_type=jnp.float32)
    rowio = jax.lax.broadcasted_iota(jnp.int32, (V, V), 0)
    colio = jax.lax.broadcasted_iota(jnp.int32, (V, V), 1)
    diag = rowio == colio
    n2col = n2all[:, 0:1]
    n2row = jnp.sum(jnp.where(diag, G, 0.0), axis=0, keepdims=True)
    d2 = n2col + n2row - 2.0 * G
    d2 = jnp.where(diag, BIG, d2)

    feat = jnp.concatenate([
        jnp.ones((V, 1), jnp.float32), nall,
        jnp.full((V, 1), ZERO_NORM, jnp.float32),
    ], axis=1)

    h = _dot(_relu(_dot(feat, Wenc1[...]) + benc1[...]), Wenc2[...]) + benc2[...]

    _i32 = lambda x: jax.lax.bitcast_convert_type(x, jnp.int32)
    _f32 = lambda x: jax.lax.bitcast_convert_type(x, jnp.float32)
    d2c = jnp.maximum(d2, jnp.float32(1e-30))
    pk = _f32((_i32(d2c) & ~jnp.int32(V - 1)) | colio)
    for k in range(K):
        kmin = _i32(jnp.min(pk, axis=1, keepdims=True))
        idx = kmin & jnp.int32(V - 1)
        d2sel = _f32(kmin & ~jnp.int32(V - 1))
        idx_ref[pl.ds(k * V, V), :] = idx
        dist_ref[pl.ds(k * V, V), :] = d2sel
        if k < K - 1:
            pk = jnp.where(colio == idx, jnp.float32(jnp.inf), pk)

    idx_all = idx_ref[...]
    cb = jax.lax.broadcasted_iota(jnp.int32, (K * V, V), 1)
    ohe = (cb == idx_all).astype(jnp.float32)

    g1 = _dot(ohe, jnp.concatenate([h, gp], axis=1))
    hs = g1[:, :LATENT]
    gsrc = g1[:, LATENT:]

    gdst = jnp.concatenate([gp] * K, axis=0)
    rel = gsrc - gdst
    dist = jnp.sqrt(dist_ref[...] + EPS)
    e_in = jnp.concatenate([rel, dist], axis=1)
    ea = _dot(_relu(_dot(e_in, We1[...]) + be1[...]), We2[...]) + be2[...]

    for r in range(2):
        if r:
            hs = _dot(ohe, h)
        msg = _relu(_dot(hs * ea, Wmsg[...]) + bmsg[...])
        agg = msg[0 * V:1 * V]
        for k in range(1, K):
            agg = agg + msg[k * V:(k + 1) * V]
        h = h + _relu(_dot(agg, Wupd[...]) + bupd[...])

    z = _dot(_relu(_dot(h, Wdec1[...]) + bdec1[...]), Wdec2[...]) + bdec2[...]
    return jnp.mean(z, axis=0, keepdims=True)


def _env_body(gp_ref, gv_ref, tp_ref,
              Wenc1, benc1, Wenc2, benc2, We1, be1, We2, be2,
              Wmsg, bmsg, Wupd, bupd, Wdec1, bdec1, Wdec2, bdec2,
              out_ref, dist_ref, idx_ref):
    for e in range(E):
        out_ref[e] = _one_env(
            gp_ref[e], gv_ref[e], tp_ref[e],
            Wenc1, benc1, Wenc2, benc2, We1, be1, We2, be2,
            Wmsg, bmsg, Wupd, bupd, Wdec1, bdec1, Wdec2, bdec2,
            dist_ref, idx_ref)


@jax.jit
def _run(gp, gv, tp, Wenc1, benc1, Wenc2, benc2, We1, be1, We2, be2,
         Wmsg, bmsg, Wupd, bupd, Wdec1, bdec1, Wdec2, bdec2):
    env_spec3 = lambda s: pl.BlockSpec((E,) + s, lambda i: (i, 0, 0))
    wspec = lambda s: pl.BlockSpec(s, lambda i: (0, 0))
    args = (Wenc1, benc1, Wenc2, benc2, We1, be1, We2, be2,
            Wmsg, bmsg, Wupd, bupd, Wdec1, bdec1, Wdec2, bdec2)
    out = pl.pallas_call(
        _env_body,
        grid=(N // E,),
        in_specs=[env_spec3((V, 3)), env_spec3((V, 3)), env_spec3((T, 3))]
                 + [wspec(a.shape) for a in args],
        out_specs=pl.BlockSpec((E, 1, LATENT), lambda i: (i, 0, 0)),
        out_shape=jax.ShapeDtypeStruct((N, 1, LATENT), jnp.float32),
        scratch_shapes=[
            pltpu.VMEM((K * V, 1), jnp.float32),
            pltpu.VMEM((K * V, 1), jnp.int32),
        ],
    )(gp, gv, tp, *args)
    return out.reshape(N, LATENT)


def kernel(glove_pos, glove_vel, ee_pos, ee_vel, target_pos,
           W_enc1, b_enc1, W_enc2, b_enc2, W_e1, b_e1, W_e2, b_e2,
           W_msg, b_msg, W_upd, b_upd, W_dec1, b_dec1, W_dec2, b_dec2):
    del ee_pos, ee_vel
    b2 = lambda b: b.reshape(1, LATENT)
    return _run(glove_pos, glove_vel, target_pos,
                W_enc1, b2(b_enc1), W_enc2, b2(b_enc2),
                W_e1, b2(b_e1), W_e2, b2(b_e2),
                W_msg, b2(b_msg), W_upd, b2(b_upd),
                W_dec1, b2(b_dec1), W_dec2, b2(b_dec2))
